# baseline (speedup 1.0000x reference)
import dataclasses
import functools

import jax
import jax.numpy as jnp
from jax import lax
from jax.experimental import pallas as pl
from jax.experimental.pallas import tpu as pltpu
from jax.experimental.pallas import tpu_sc as plsc

_V = 1000
_B = 16384
_L = 50

_NC = 2
_NS = 16
_NW = _NC * _NS

_H = 4
_BC = _B // _H
_BPW = _BC // _NW
_G = 4
_CHUNK = _G * _L * _L
_ITERS = _CHUNK // 16
_STEPS = _BPW // _G

_NB = 256

_CP = pltpu.CompilerParams()
if "needs_layout_passes" in pltpu.CompilerParams.__dataclass_fields__:
    _CP = dataclasses.replace(_CP, needs_layout_passes=False)


def _make_sc_chunk(chunk):
    ibase0 = chunk * _BC * _L

    @functools.partial(
        pl.kernel,
        out_type=jax.ShapeDtypeStruct((_BC, _L, _L), jnp.float32),
        mesh=plsc.VectorSubcoreMesh(core_axis_name="c", subcore_axis_name="s"),
        compiler_params=_CP,
        scratch_types=[
            pltpu.VMEM((_G * _L,), jnp.int32),
            pltpu.VMEM((_CHUNK,), jnp.int32),
            pltpu.VMEM((_CHUNK,), jnp.int32),
            pltpu.VMEM((_CHUNK,), jnp.int32),
            pltpu.VMEM((_CHUNK,), jnp.float32),
            pltpu.VMEM((_CHUNK,), jnp.float32),
            pltpu.VMEM((_L, _L), jnp.float32),
            pltpu.VMEM((_L, _L), jnp.float32),
            pltpu.VMEM_SHARED((_V * _V,), jnp.float32),
            pltpu.SemaphoreType.DMA,
            pltpu.SemaphoreType.DMA,
            pltpu.SemaphoreType.DMA,
            pltpu.SemaphoreType.DMA,
            pltpu.SemaphoreType.DMA,
        ],
    )
    def sc_chunk(wflat_hbm, idx_hbm, bt_hbm, out_hbm,
                 idx_v, bt_v, fbuf0, fbuf1, vals0, vals1, obuf0, obuf1,
                 w_sh, sw, sg0, sg1, so0, so1):
        sid = lax.axis_index("s")
        wid = sid * _NC + lax.axis_index("c")
        ibase = ibase0 + wid * _BPW * _L
        bbase = wid * _BPW

        @pl.when(sid == 0)
        def _():
            pltpu.async_copy(wflat_hbm, w_sh, sw).wait()

        pltpu.sync_copy(bt_hbm, bt_v)
        plsc.subcore_barrier()

        iota16 = lax.iota(jnp.int32, 16)
        tailmask = iota16 < 2

        def build(step, fbuf):
            pltpu.sync_copy(
                idx_hbm.at[pl.ds(ibase + step * (_G * _L), _G * _L)], idx_v)

            @pl.loop(0, _ITERS)
            def _(k):
                u = k * 16
                t = bt_v[pl.ds(u, 16)]
                r = plsc.load_gather(idx_v, [t >> 8])
                c = plsc.load_gather(idx_v, [t & 255])
                fbuf[pl.ds(u, 16)] = r * _V + c

        def start_gather(fbuf, vals, sem):
            pltpu.async_copy(w_sh.at[fbuf], vals, sem)

        def wait_gather(fbuf, vals, sem):
            pltpu.make_async_copy(w_sh.at[fbuf], vals, sem).wait()

        def reformat(vals, g, obuf):
            @pl.loop(0, _L)
            def _(m):
                src = g * (_L * _L) + m * _L
                for q in range(3):
                    v = plsc.load_gather(vals, [iota16 + (src + q * 16)])
                    obuf[m, pl.ds(q * 16, 16)] = v
                v = plsc.load_gather(vals, [iota16 + (src + 48)],
                                     mask=tailmask)
                plsc.store_scatter(obuf, [jnp.full((16,), m, jnp.int32),
                                          iota16 + 48], v, mask=tailmask)

        def out_dst(step, g):
            return out_hbm.at[bbase + step * _G + g]

        def emit_chunk(step, vals, first):
            for g in range(_G):
                obuf = obuf0 if g % 2 == 0 else obuf1
                sem = so0 if g % 2 == 0 else so1
                if g >= 2:
                    pltpu.make_async_copy(
                        obuf, out_dst(step, g - 2), sem).wait()
                else:
                    @pl.when(jnp.logical_not(first))
                    def _():
                        pltpu.make_async_copy(
                            obuf, out_dst(step - 1, g + 2), sem).wait()
                reformat(vals, g, obuf)
                pltpu.async_copy(obuf, out_dst(step, g), sem)

        build(0, fbuf0)
        start_gather(fbuf0, vals0, sg0)

        @pl.loop(0, _STEPS // 2)
        def _(h):
            s0 = h * 2

            build(s0 + 1, fbuf1)
            start_gather(fbuf1, vals1, sg1)
            wait_gather(fbuf0, vals0, sg0)
            emit_chunk(s0, vals0, h == 0)

            @pl.when(h < _STEPS // 2 - 1)
            def _():
                build(s0 + 2, fbuf0)
                start_gather(fbuf0, vals0, sg0)

            wait_gather(fbuf1, vals1, sg1)
            emit_chunk(s0 + 1, vals1, False)

        pltpu.make_async_copy(obuf0, out_dst(_STEPS - 1, 2), so0).wait()
        pltpu.make_async_copy(obuf1, out_dst(_STEPS - 1, 3), so1).wait()

    return sc_chunk


_SC_CHUNKS = [_make_sc_chunk(c) for c in range(_H)]


def _tc_copy_first(part):
    def body(x_ref, o_ref):
        o_ref[...] = x_ref[...]

    return pl.pallas_call(
        body,
        grid=(_BC // _NB,),
        in_specs=[pl.BlockSpec((_NB, _L, _L), lambda i: (i, 0, 0))],
        out_specs=pl.BlockSpec((_NB, _L, _L), lambda i: (i, 0, 0)),
        out_shape=jax.ShapeDtypeStruct((_B, _L, _L), jnp.float32),
    )(part)


def _tc_copy_chunk(chunk, part, acc):
    def body(x_ref, a_ref, o_ref):
        o_ref[...] = x_ref[...]

    base = chunk * (_BC // _NB)
    return pl.pallas_call(
        body,
        grid=(_BC // _NB,),
        in_specs=[
            pl.BlockSpec((_NB, _L, _L), lambda i: (i, 0, 0)),
            pl.BlockSpec((_NB, _L, _L), lambda i, b=base: (b + i, 0, 0)),
        ],
        out_specs=pl.BlockSpec((_NB, _L, _L), lambda i, b=base: (b + i, 0, 0)),
        out_shape=jax.ShapeDtypeStruct((_B, _L, _L), jnp.float32),
        input_output_aliases={1: 0},
    )(part, acc)


def _packed_table():
    u = jnp.arange(_CHUNK, dtype=jnp.int32)
    b_loc = u // (_L * _L)
    k = u % (_L * _L)
    bi = b_loc * _L + k // _L
    bj = b_loc * _L + k % _L
    return bi * 256 + bj


def kernel(indices, innerVars):
    bt = _packed_table()
    wflat = innerVars.reshape(_V * _V)
    idxflat = indices.reshape(_B * _L)
    parts = [sc(wflat, idxflat, bt) for sc in _SC_CHUNKS]
    out = _tc_copy_first(parts[0])
    for c in range(1, _H):
        out = _tc_copy_chunk(c, parts[c], out)
    return out


# speedup vs baseline: 1.1626x; 1.1626x over previous
import dataclasses
import functools

import jax
import jax.numpy as jnp
from jax import lax
from jax.experimental import pallas as pl
from jax.experimental.pallas import tpu as pltpu
from jax.experimental.pallas import tpu_sc as plsc

_V = 1000
_B = 16384
_L = 50

_NC = 2
_NS = 16
_NW = _NC * _NS

_BPW = _B // _NW
_G = 4
_CHUNK = _G * _L * _L
_ITERS = _CHUNK // 16
_STEPS = _BPW // _G

_CP = pltpu.CompilerParams()
if "needs_layout_passes" in pltpu.CompilerParams.__dataclass_fields__:
    _CP = dataclasses.replace(_CP, needs_layout_passes=False)


@functools.partial(
    pl.kernel,
    out_type=jax.ShapeDtypeStruct((_B, _L, _L), jnp.float32),
    mesh=plsc.VectorSubcoreMesh(core_axis_name="c", subcore_axis_name="s"),
    compiler_params=_CP,
    scratch_types=[
        pltpu.VMEM((_G * _L,), jnp.int32),
        pltpu.VMEM((_CHUNK,), jnp.int32),
        pltpu.VMEM((_CHUNK,), jnp.int32),
        pltpu.VMEM((_CHUNK,), jnp.int32),
        pltpu.VMEM((_CHUNK + 16,), jnp.float32),
        pltpu.VMEM((_CHUNK + 16,), jnp.float32),
        pltpu.VMEM((_L, _L), jnp.float32),
        pltpu.VMEM((_L, _L), jnp.float32),
        pltpu.VMEM_SHARED((_V * _V,), jnp.float32),
        pltpu.SemaphoreType.DMA,
        pltpu.SemaphoreType.DMA,
        pltpu.SemaphoreType.DMA,
        pltpu.SemaphoreType.DMA,
        pltpu.SemaphoreType.DMA,
    ],
)
def _sc_kmat(wflat_hbm, idx_hbm, bt_hbm, out_hbm,
             idx_v, bt_v, fbuf0, fbuf1, vals0, vals1, obuf0, obuf1,
             w_sh, sw, sg0, sg1, so0, so1):
    sid = lax.axis_index("s")
    wid = sid * _NC + lax.axis_index("c")
    ibase = wid * _BPW * _L
    bbase = wid * _BPW

    @pl.when(sid == 0)
    def _():
        pltpu.async_copy(wflat_hbm, w_sh, sw).wait()

    pltpu.sync_copy(bt_hbm, bt_v)
    plsc.subcore_barrier()

    iota16 = lax.iota(jnp.int32, 16)

    def build(step, fbuf):
        pltpu.sync_copy(idx_hbm.at[pl.ds(ibase + step * (_G * _L), _G * _L)],
                        idx_v)

        @pl.loop(0, _ITERS)
        def _(k):
            u = k * 16
            t = bt_v[pl.ds(u, 16)]
            r = plsc.load_gather(idx_v, [t >> 8])
            c = plsc.load_gather(idx_v, [t & 255])
            fbuf[pl.ds(u, 16)] = r * _V + c

    def start_gather(fbuf, vals, sem):
        pltpu.async_copy(w_sh.at[fbuf], vals.at[pl.ds(0, _CHUNK)], sem)

    def wait_gather(fbuf, vals, sem):
        pltpu.make_async_copy(
            w_sh.at[fbuf], vals.at[pl.ds(0, _CHUNK)], sem).wait()

    tailmask = iota16 < 2

    def reformat(vals, g, obuf):
        @pl.loop(0, _L)
        def _(m):
            src = g * (_L * _L) + m * _L
            for q in range(3):
                v = plsc.load_gather(vals, [iota16 + (src + q * 16)])
                obuf[m, pl.ds(q * 16, 16)] = v
            v = plsc.load_gather(vals, [iota16 + (src + 48)], mask=tailmask)
            plsc.store_scatter(obuf, [jnp.full((16,), m, jnp.int32),
                                      iota16 + 48], v, mask=tailmask)

    def out_dst(step, g):
        b = bbase + step * _G + g
        return out_hbm.at[b]

    def emit_chunk(step, vals, first):
        for g in range(_G):
            obuf = obuf0 if g % 2 == 0 else obuf1
            sem = so0 if g % 2 == 0 else so1
            if g >= 2:
                pltpu.make_async_copy(obuf, out_dst(step, g - 2), sem).wait()
            else:
                @pl.when(jnp.logical_not(first))
                def _():
                    pltpu.make_async_copy(
                        obuf, out_dst(step - 1, g + 2), sem).wait()
            reformat(vals, g, obuf)
            pltpu.async_copy(obuf, out_dst(step, g), sem)

    build(0, fbuf0)
    start_gather(fbuf0, vals0, sg0)

    @pl.loop(0, _STEPS // 2)
    def _(h):
        s0 = h * 2

        build(s0 + 1, fbuf1)
        start_gather(fbuf1, vals1, sg1)
        wait_gather(fbuf0, vals0, sg0)
        emit_chunk(s0, vals0, h == 0)

        @pl.when(h < _STEPS // 2 - 1)
        def _():
            build(s0 + 2, fbuf0)
            start_gather(fbuf0, vals0, sg0)

        wait_gather(fbuf1, vals1, sg1)
        emit_chunk(s0 + 1, vals1, False)

    pltpu.make_async_copy(obuf0, out_dst(_STEPS - 1, 2), so0).wait()
    pltpu.make_async_copy(obuf1, out_dst(_STEPS - 1, 3), so1).wait()


def _packed_table():
    u = jnp.arange(_CHUNK, dtype=jnp.int32)
    b_loc = u // (_L * _L)
    k = u % (_L * _L)
    bi = b_loc * _L + k // _L
    bj = b_loc * _L + k % _L
    return bi * 256 + bj


def kernel(indices, innerVars):
    bt = _packed_table()
    return _sc_kmat(innerVars.reshape(_V * _V), indices.reshape(_B * _L), bt)
